# baseline (speedup 1.0000x reference)
import functools

import jax
import jax.numpy as jnp
from jax import lax
from jax.experimental import pallas as pl
from jax.experimental.pallas import tpu as pltpu
from jax.experimental.pallas import tpu_sc as plsc

NN = 50000
EE = 800000
LANE = 16
NC = 2
NS = 16
CHUNK = 128
EPAD = 819200
EPT16 = EPAD // NS
EPT32 = EPAD // (NC * NS)
NPAD = 50048
NZT = NPAD // NS
RB = 1000
NB = NN // RB
EPS = 1e-5

_SC_PARAMS = pltpu.CompilerParams(use_tc_tiling_on_sc=False)


@functools.cache
def _mesh():
    return plsc.VectorSubcoreMesh(core_axis_name="c", subcore_axis_name="s",
                                  num_cores=NC, num_subcores=NS)


@functools.cache
def _deg_kernel():
    scratch = [
        pltpu.VMEM_SHARED((NPAD, LANE), jnp.float32),
        pltpu.VMEM((CHUNK,), jnp.int32),
        pltpu.VMEM((CHUNK, LANE), jnp.float32),
    ]
    out_type = (jax.ShapeDtypeStruct((NPAD, 128), jnp.float32),) * 2

    @functools.partial(pl.kernel, out_type=out_type, mesh=_mesh(),
                       scratch_types=scratch, compiler_params=_SC_PARAMS)
    def k(dst_hbm, zeros_hbm, ones_hbm, out_a, out_b, acc, dstv, ones_v):
        cid = lax.axis_index("c")
        sid = lax.axis_index("s")
        z0 = sid * NZT
        pltpu.sync_copy(zeros_hbm.at[pl.ds(z0, NZT), pl.ds(0, LANE)],
                        acc.at[pl.ds(z0, NZT)])
        pltpu.sync_copy(ones_hbm, ones_v)
        plsc.subcore_barrier()
        wid = cid * NS + sid

        def body(i, carry):
            base = wid * EPT32 + i * CHUNK
            pltpu.sync_copy(dst_hbm.at[pl.ds(base, CHUNK)], dstv)
            pltpu.sync_copy(ones_v, acc.at[dstv], add=True)
            return carry

        lax.fori_loop(0, EPT32 // CHUNK, body, 0)
        plsc.subcore_barrier()

        @pl.when(cid == 0)
        def _():
            pltpu.sync_copy(acc.at[pl.ds(z0, NZT)],
                            out_a.at[pl.ds(z0, NZT), pl.ds(0, LANE)])

        @pl.when(cid == 1)
        def _():
            pltpu.sync_copy(acc.at[pl.ds(z0, NZT)],
                            out_b.at[pl.ds(z0, NZT), pl.ds(0, LANE)])

    return k


WPAIR = 2 * LANE


@functools.cache
def _gather_scatter_kernel(C):
    KC = (C + 7) // 8
    NPR = C // 2
    split_edges = NPR == 1
    nouts = 2 if split_edges else KC
    NBUF = 4
    scratch = (
        [pltpu.VMEM_SHARED((NPAD, WPAIR), jnp.float32)]
        + [pltpu.VMEM((CHUNK,), jnp.int32) for _ in range(3 * NBUF)]
        + [pltpu.VMEM((CHUNK, WPAIR), jnp.float32) for _ in range(NBUF)]
        + [pltpu.SemaphoreType.DMA for _ in range(2 * NBUF)]
    )
    out_type = tuple(jax.ShapeDtypeStruct((NPAD, 128), jnp.float32)
                     for _ in range(nouts))

    @functools.partial(pl.kernel, out_type=out_type, mesh=_mesh(),
                       scratch_types=scratch, compiler_params=_SC_PARAMS)
    def k(src_hbm, dst_hbm, zeros_hbm, *rest):
        tables = rest[:KC]
        outs = rest[KC:KC + nouts]
        scr = rest[KC + nouts:]
        acc = scr[0]
        srcvs = scr[1:1 + NBUF]
        dstvs = scr[1 + NBUF:1 + 3 * NBUF]
        rowss = scr[1 + 3 * NBUF:1 + 4 * NBUF]
        sems = scr[1 + 4 * NBUF:1 + 5 * NBUF]
        ssems = scr[1 + 5 * NBUF:1 + 6 * NBUF]
        cid = lax.axis_index("c")
        sid = lax.axis_index("s")
        z0 = sid * NZT

        def run_pair(table, out, jp, tbase, nchunks):
            pltpu.sync_copy(zeros_hbm.at[pl.ds(z0, NZT)],
                            acc.at[pl.ds(z0, NZT)])
            plsc.subcore_barrier()

            nq = nchunks // NBUF

            def load_and_fire(ci, b, u, first):
                base = tbase + ci * CHUNK
                sv, dv = srcvs[b], dstvs[u]
                pltpu.sync_copy(src_hbm.at[pl.ds(base, CHUNK)], sv)
                pltpu.sync_copy(dst_hbm.at[pl.ds(base, CHUNK)], dv)
                for q in range(CHUNK // LANE):
                    sv[pl.ds(q * LANE, LANE)] = (
                        sv[pl.ds(q * LANE, LANE)] * 4 + jp)
                if not first:
                    pltpu.make_async_copy(
                        rowss[b], acc.at[dstvs[u]], ssems[b]).wait()
                pltpu.async_copy(table.at[sv], rowss[b], sems[b])

            for b in range(NBUF):
                load_and_fire(b, b, b, True)

            def body(P, carry):
                for half in range(2):
                    p = 2 * P + half
                    ds = half * NBUF
                    do = (1 - half) * NBUF
                    for b in range(NBUF):
                        pltpu.make_async_copy(
                            table.at[srcvs[b]], rowss[b], sems[b]).wait()
                        pltpu.async_copy(rowss[b], acc.at[dstvs[ds + b]],
                                         ssems[b], add=True)

                        @pl.when(p + 1 < nq)
                        def _(b=b, p=p, do=do):
                            load_and_fire(NBUF * (p + 1) + b, b, do + b,
                                          False)

                        @pl.when(p + 1 >= nq)
                        def _(b=b, ds=ds):
                            pltpu.make_async_copy(
                                rowss[b], acc.at[dstvs[ds + b]],
                                ssems[b]).wait()
                return carry

            lax.fori_loop(0, nq // 2, body, 0)
            plsc.subcore_barrier()
            pltpu.sync_copy(acc.at[pl.ds(z0, NZT)],
                            out.at[pl.ds(z0, NZT), pl.ds(jp * WPAIR, WPAIR)])
            plsc.subcore_barrier()

        if split_edges:
            for half in range(NC):
                @pl.when(cid == half)
                def _(half=half):
                    run_pair(tables[0], outs[half], 0,
                             (half * NS + sid) * EPT32, EPT32 // CHUNK)
        else:
            for half in range(NC):
                @pl.when(cid == half)
                def _(half=half):
                    for p in range(half, NPR, NC):
                        run_pair(tables[p // 4], outs[p // 4], p % 4,
                                 sid * EPT16, EPT16 // CHUNK)

    return k


def _dinv_of(da_ref, db_ref):
    deg = da_ref[:, 0:1] + db_ref[:, 0:1] + 1.0
    return lax.rsqrt(deg)


def _pad128(h):
    dout = h.shape[-1]
    if dout % 128 == 0:
        return h
    return jnp.concatenate(
        [h, jnp.zeros((h.shape[0], 128 - dout % 128), jnp.float32)], axis=1)


def _g_chunk_specs(kc):
    return tuple(pl.BlockSpec((RB, 128), lambda i: (i, 0)) for _ in range(kc))


def _g_chunk_shapes(kc):
    return tuple(jax.ShapeDtypeStruct((NPAD, 128), jnp.float32)
                 for _ in range(kc))


@functools.cache
def _first_layer_call(dout):
    kc = (dout + 127) // 128

    def body(x_ref, w_ref, da_ref, db_ref, *outs):
        h = jnp.dot(x_ref[...], w_ref[...],
                    preferred_element_type=jnp.float32)
        g = _pad128(h * _dinv_of(da_ref, db_ref))
        for q, o in enumerate(outs):
            o[...] = g[:, q * 128:(q + 1) * 128]

    return pl.pallas_call(
        body,
        grid=(NB,),
        in_specs=[
            pl.BlockSpec((RB, 8), lambda i: (i, 0)),
            pl.BlockSpec((8, dout), lambda i: (0, 0)),
            pl.BlockSpec((RB, 128), lambda i: (i, 0)),
            pl.BlockSpec((RB, 128), lambda i: (i, 0)),
        ],
        out_specs=_g_chunk_specs(kc),
        out_shape=_g_chunk_shapes(kc),
    )


@functools.cache
def _mid_layer_call(din, dout):
    kc = (dout + 127) // 128

    def body(y_ref, st_ref, w_ref, da_ref, db_ref, *outs):
        m = st_ref[0:1, :] / NN
        v = st_ref[1:2, :] / NN - m * m
        z = jax.nn.relu((y_ref[...] - m) * lax.rsqrt(v + EPS))
        h = jnp.dot(z, w_ref[...], preferred_element_type=jnp.float32)
        g = _pad128(h * _dinv_of(da_ref, db_ref))
        for q, o in enumerate(outs):
            o[...] = g[:, q * 128:(q + 1) * 128]

    return pl.pallas_call(
        body,
        grid=(NB,),
        in_specs=[
            pl.BlockSpec((RB, din), lambda i: (i, 0)),
            pl.BlockSpec((2, din), lambda i: (0, 0)),
            pl.BlockSpec((din, dout), lambda i: (0, 0)),
            pl.BlockSpec((RB, 128), lambda i: (i, 0)),
            pl.BlockSpec((RB, 128), lambda i: (i, 0)),
        ],
        out_specs=_g_chunk_specs(kc),
        out_shape=_g_chunk_shapes(kc),
    )


@functools.cache
def _combine_call(dout, ns):
    kc = (dout + 127) // 128
    spc = ns // kc

    def body(*refs):
        s_refs = refs[:ns]
        g_refs = refs[ns:ns + kc]
        da_ref, db_ref = refs[ns + kc:ns + kc + 2]
        y_ref, st_ref = refs[ns + kc + 2:ns + kc + 4]
        acc = refs[ns + kc + 4]
        i = pl.program_id(0)
        dinv = _dinv_of(da_ref, db_ref)
        parts = []
        for q in range(kc):
            s = s_refs[q * spc][...]
            for w in range(1, spc):
                s = s + s_refs[q * spc + w][...]
            parts.append((s + g_refs[q][...]) * dinv)
        y = jnp.concatenate(parts, axis=1)[:, :dout]
        y_ref[...] = y

        @pl.when(i == 0)
        def _():
            acc[...] = jnp.zeros((2, dout), jnp.float32)

        s1 = jnp.sum(y, axis=0, keepdims=True)
        s2 = jnp.sum(y * y, axis=0, keepdims=True)
        acc[...] = acc[...] + jnp.concatenate([s1, s2], axis=0)

        @pl.when(i == NB - 1)
        def _():
            st_ref[...] = acc[...]

    return pl.pallas_call(
        body,
        grid=(NB,),
        in_specs=list(_g_chunk_specs(ns)) + list(_g_chunk_specs(kc))
                 + [pl.BlockSpec((RB, 128), lambda i: (i, 0)),
                    pl.BlockSpec((RB, 128), lambda i: (i, 0))],
        out_specs=(pl.BlockSpec((RB, dout), lambda i: (i, 0)),
                   pl.BlockSpec((2, dout), lambda i: (0, 0))),
        out_shape=(jax.ShapeDtypeStruct((NN, dout), jnp.float32),
                   jax.ShapeDtypeStruct((2, dout), jnp.float32)),
        scratch_shapes=[pltpu.VMEM((2, dout), jnp.float32)],
    )


@functools.cache
def _final_call(dout):
    def body(y_ref, st_ref, o_ref):
        m = st_ref[0:1, :] / NN
        v = st_ref[1:2, :] / NN - m * m
        o_ref[...] = jax.nn.relu((y_ref[...] - m) * lax.rsqrt(v + EPS))

    return pl.pallas_call(
        body,
        grid=(NB,),
        in_specs=[
            pl.BlockSpec((RB, dout), lambda i: (i, 0)),
            pl.BlockSpec((2, dout), lambda i: (0, 0)),
        ],
        out_specs=pl.BlockSpec((RB, dout), lambda i: (i, 0)),
        out_shape=jax.ShapeDtypeStruct((NN, dout), jnp.float32),
    )


@jax.jit
def _run(x, edge_index, W0, W1, W2, W3):
    src = jnp.concatenate(
        [edge_index[0], jnp.zeros((EPAD - EE,), jnp.int32)])
    dst = jnp.concatenate(
        [edge_index[1], jnp.full((EPAD - EE,), NN, jnp.int32)])
    zeros = jnp.zeros((NPAD, WPAIR), jnp.float32)
    ones = jnp.ones((CHUNK, LANE), jnp.float32)

    deg_a, deg_b = _deg_kernel()(dst, zeros, ones)

    xp = jnp.pad(x, ((0, 0), (0, 1)))
    w0p = jnp.pad(W0, ((0, 1), (0, 0)))
    Ws = [w0p, W1, W2, W3]
    douts = [32, 64, 128, 256]

    g = _first_layer_call(douts[0])(xp, Ws[0], deg_a, deg_b)
    for l in range(4):
        dout = douts[l]
        tall = tuple(c.reshape(4 * NPAD, WPAIR) for c in g)
        S = _gather_scatter_kernel(dout // LANE)(src, dst, zeros, *tall)
        y, st = _combine_call(dout, len(S))(*S, *g, deg_a, deg_b)
        if l < 3:
            g = _mid_layer_call(dout, douts[l + 1])(
                y, st, Ws[l + 1], deg_a, deg_b)
    return _final_call(douts[3])(y, st)


def kernel(x, edge_index, W0, b0, W1, b1, W2, b2, W3, b3):
    return _run(x, edge_index, W0, W1, W2, W3)


# speedup vs baseline: 1.2810x; 1.2810x over previous
import functools

import jax
import jax.numpy as jnp
from jax import lax
from jax.experimental import pallas as pl
from jax.experimental.pallas import tpu as pltpu
from jax.experimental.pallas import tpu_sc as plsc

NN = 50000
EE = 800000
LANE = 16
NC = 2
NS = 16
CHUNK = 128
EPAD = 802816
EPT16 = EPAD // NS
EPT32 = EPAD // (NC * NS)
NPAD = 50048
NZT = NPAD // NS
RB = 1000
NB = NN // RB
EPS = 1e-5

_SC_PARAMS = pltpu.CompilerParams(use_tc_tiling_on_sc=False)


@functools.cache
def _mesh():
    return plsc.VectorSubcoreMesh(core_axis_name="c", subcore_axis_name="s",
                                  num_cores=NC, num_subcores=NS)


@functools.cache
def _deg_kernel():
    scratch = [
        pltpu.VMEM_SHARED((NPAD, LANE), jnp.float32),
        pltpu.VMEM((CHUNK,), jnp.int32),
        pltpu.VMEM((CHUNK, LANE), jnp.float32),
    ]
    out_type = (jax.ShapeDtypeStruct((NPAD, 128), jnp.float32),) * 2

    @functools.partial(pl.kernel, out_type=out_type, mesh=_mesh(),
                       scratch_types=scratch, compiler_params=_SC_PARAMS)
    def k(dst_hbm, zeros_hbm, ones_hbm, out_a, out_b, acc, dstv, ones_v):
        cid = lax.axis_index("c")
        sid = lax.axis_index("s")
        z0 = sid * NZT
        pltpu.sync_copy(zeros_hbm.at[pl.ds(z0, NZT), pl.ds(0, LANE)],
                        acc.at[pl.ds(z0, NZT)])
        pltpu.sync_copy(ones_hbm, ones_v)
        plsc.subcore_barrier()
        wid = cid * NS + sid

        def body(i, carry):
            base = wid * EPT32 + i * CHUNK
            pltpu.sync_copy(dst_hbm.at[pl.ds(base, CHUNK)], dstv)
            pltpu.sync_copy(ones_v, acc.at[dstv], add=True)
            return carry

        lax.fori_loop(0, EPT32 // CHUNK, body, 0)
        plsc.subcore_barrier()

        @pl.when(cid == 0)
        def _():
            pltpu.sync_copy(acc.at[pl.ds(z0, NZT)],
                            out_a.at[pl.ds(z0, NZT), pl.ds(0, LANE)])

        @pl.when(cid == 1)
        def _():
            pltpu.sync_copy(acc.at[pl.ds(z0, NZT)],
                            out_b.at[pl.ds(z0, NZT), pl.ds(0, LANE)])

    return k


WPAIR = 2 * LANE


@functools.cache
def _gather_scatter_kernel(C):
    KC = (C + 7) // 8
    NPR = C // 2
    split_edges = NPR == 1
    nouts = 2 if split_edges else KC
    NBUF = 4
    scratch = (
        [pltpu.VMEM_SHARED((NPAD, WPAIR), jnp.float32)]
        + [pltpu.VMEM((CHUNK,), jnp.int32) for _ in range(2 * NBUF)]
        + [pltpu.VMEM((CHUNK, WPAIR), jnp.float32) for _ in range(NBUF)]
        + [pltpu.SemaphoreType.DMA for _ in range(NBUF)]
    )
    out_type = tuple(jax.ShapeDtypeStruct((NPAD, 128), jnp.float32)
                     for _ in range(nouts))

    @functools.partial(pl.kernel, out_type=out_type, mesh=_mesh(),
                       scratch_types=scratch, compiler_params=_SC_PARAMS)
    def k(src_hbm, dst_hbm, zeros_hbm, *rest):
        tables = rest[:KC]
        outs = rest[KC:KC + nouts]
        scr = rest[KC + nouts:]
        acc = scr[0]
        srcvs = scr[1:1 + NBUF]
        dstvs = scr[1 + NBUF:1 + 2 * NBUF]
        rowss = scr[1 + 2 * NBUF:1 + 3 * NBUF]
        sems = scr[1 + 3 * NBUF:1 + 4 * NBUF]
        cid = lax.axis_index("c")
        sid = lax.axis_index("s")
        z0 = sid * NZT

        def run_pair(table, out, jp, tbase, nchunks):
            pltpu.sync_copy(zeros_hbm.at[pl.ds(z0, NZT)],
                            acc.at[pl.ds(z0, NZT)])
            plsc.subcore_barrier()

            def load_and_fire(ci, b):
                base = tbase + ci * CHUNK
                sv, dv = srcvs[b], dstvs[b]
                pltpu.sync_copy(src_hbm.at[pl.ds(base, CHUNK)], sv)
                pltpu.sync_copy(dst_hbm.at[pl.ds(base, CHUNK)], dv)
                for q in range(CHUNK // LANE):
                    sv[pl.ds(q * LANE, LANE)] = (
                        sv[pl.ds(q * LANE, LANE)] * 4 + jp)
                pltpu.async_copy(table.at[sv], rowss[b], sems[b])

            for b in range(NBUF):
                load_and_fire(b, b)
            nq = nchunks // NBUF

            def body(p, carry):
                for b in range(NBUF):
                    pltpu.make_async_copy(
                        table.at[srcvs[b]], rowss[b], sems[b]).wait()
                    pltpu.sync_copy(rowss[b], acc.at[dstvs[b]], add=True)

                    @pl.when(p + 1 < nq)
                    def _(b=b):
                        load_and_fire(NBUF * (p + 1) + b, b)
                return carry

            lax.fori_loop(0, nq, body, 0)
            plsc.subcore_barrier()
            pltpu.sync_copy(acc.at[pl.ds(z0, NZT)],
                            out.at[pl.ds(z0, NZT), pl.ds(jp * WPAIR, WPAIR)])
            plsc.subcore_barrier()

        if split_edges:
            for half in range(NC):
                @pl.when(cid == half)
                def _(half=half):
                    run_pair(tables[0], outs[half], 0,
                             (half * NS + sid) * EPT32, EPT32 // CHUNK)
        else:
            for half in range(NC):
                @pl.when(cid == half)
                def _(half=half):
                    for p in range(half, NPR, NC):
                        run_pair(tables[p // 4], outs[p // 4], p % 4,
                                 sid * EPT16, EPT16 // CHUNK)

    return k


def _dinv_of(da_ref, db_ref):
    deg = da_ref[:, 0:1] + db_ref[:, 0:1] + 1.0
    return lax.rsqrt(deg)


def _pad128(h):
    dout = h.shape[-1]
    if dout % 128 == 0:
        return h
    return jnp.concatenate(
        [h, jnp.zeros((h.shape[0], 128 - dout % 128), jnp.float32)], axis=1)


def _g_chunk_specs(kc):
    return tuple(pl.BlockSpec((RB, 128), lambda i: (i, 0)) for _ in range(kc))


def _g_chunk_shapes(kc):
    return tuple(jax.ShapeDtypeStruct((NPAD, 128), jnp.float32)
                 for _ in range(kc))


@functools.cache
def _first_layer_call(dout):
    kc = (dout + 127) // 128

    def body(x_ref, w_ref, da_ref, db_ref, *outs):
        h = jnp.dot(x_ref[...], w_ref[...],
                    preferred_element_type=jnp.float32)
        g = _pad128(h * _dinv_of(da_ref, db_ref))
        for q, o in enumerate(outs):
            o[...] = g[:, q * 128:(q + 1) * 128]

    return pl.pallas_call(
        body,
        grid=(NB,),
        in_specs=[
            pl.BlockSpec((RB, 8), lambda i: (i, 0)),
            pl.BlockSpec((8, dout), lambda i: (0, 0)),
            pl.BlockSpec((RB, 128), lambda i: (i, 0)),
            pl.BlockSpec((RB, 128), lambda i: (i, 0)),
        ],
        out_specs=_g_chunk_specs(kc),
        out_shape=_g_chunk_shapes(kc),
    )


@functools.cache
def _mid_layer_call(din, dout):
    kc = (dout + 127) // 128

    def body(y_ref, st_ref, w_ref, da_ref, db_ref, *outs):
        m = st_ref[0:1, :] / NN
        v = st_ref[1:2, :] / NN - m * m
        z = jax.nn.relu((y_ref[...] - m) * lax.rsqrt(v + EPS))
        h = jnp.dot(z, w_ref[...], preferred_element_type=jnp.float32)
        g = _pad128(h * _dinv_of(da_ref, db_ref))
        for q, o in enumerate(outs):
            o[...] = g[:, q * 128:(q + 1) * 128]

    return pl.pallas_call(
        body,
        grid=(NB,),
        in_specs=[
            pl.BlockSpec((RB, din), lambda i: (i, 0)),
            pl.BlockSpec((2, din), lambda i: (0, 0)),
            pl.BlockSpec((din, dout), lambda i: (0, 0)),
            pl.BlockSpec((RB, 128), lambda i: (i, 0)),
            pl.BlockSpec((RB, 128), lambda i: (i, 0)),
        ],
        out_specs=_g_chunk_specs(kc),
        out_shape=_g_chunk_shapes(kc),
    )


@functools.cache
def _combine_call(dout, ns):
    kc = (dout + 127) // 128
    spc = ns // kc

    def body(*refs):
        s_refs = refs[:ns]
        g_refs = refs[ns:ns + kc]
        da_ref, db_ref = refs[ns + kc:ns + kc + 2]
        y_ref, st_ref = refs[ns + kc + 2:ns + kc + 4]
        acc = refs[ns + kc + 4]
        i = pl.program_id(0)
        dinv = _dinv_of(da_ref, db_ref)
        parts = []
        for q in range(kc):
            s = s_refs[q * spc][...]
            for w in range(1, spc):
                s = s + s_refs[q * spc + w][...]
            parts.append((s + g_refs[q][...]) * dinv)
        y = jnp.concatenate(parts, axis=1)[:, :dout]
        y_ref[...] = y

        @pl.when(i == 0)
        def _():
            acc[...] = jnp.zeros((2, dout), jnp.float32)

        s1 = jnp.sum(y, axis=0, keepdims=True)
        s2 = jnp.sum(y * y, axis=0, keepdims=True)
        acc[...] = acc[...] + jnp.concatenate([s1, s2], axis=0)

        @pl.when(i == NB - 1)
        def _():
            st_ref[...] = acc[...]

    return pl.pallas_call(
        body,
        grid=(NB,),
        in_specs=list(_g_chunk_specs(ns)) + list(_g_chunk_specs(kc))
                 + [pl.BlockSpec((RB, 128), lambda i: (i, 0)),
                    pl.BlockSpec((RB, 128), lambda i: (i, 0))],
        out_specs=(pl.BlockSpec((RB, dout), lambda i: (i, 0)),
                   pl.BlockSpec((2, dout), lambda i: (0, 0))),
        out_shape=(jax.ShapeDtypeStruct((NN, dout), jnp.float32),
                   jax.ShapeDtypeStruct((2, dout), jnp.float32)),
        scratch_shapes=[pltpu.VMEM((2, dout), jnp.float32)],
    )


@functools.cache
def _final_call(dout):
    def body(y_ref, st_ref, o_ref):
        m = st_ref[0:1, :] / NN
        v = st_ref[1:2, :] / NN - m * m
        o_ref[...] = jax.nn.relu((y_ref[...] - m) * lax.rsqrt(v + EPS))

    return pl.pallas_call(
        body,
        grid=(NB,),
        in_specs=[
            pl.BlockSpec((RB, dout), lambda i: (i, 0)),
            pl.BlockSpec((2, dout), lambda i: (0, 0)),
        ],
        out_specs=pl.BlockSpec((RB, dout), lambda i: (i, 0)),
        out_shape=jax.ShapeDtypeStruct((NN, dout), jnp.float32),
    )


@jax.jit
def _run(x, edge_index, W0, W1, W2, W3):
    src = jnp.concatenate(
        [edge_index[0], jnp.zeros((EPAD - EE,), jnp.int32)])
    dst = jnp.concatenate(
        [edge_index[1], jnp.full((EPAD - EE,), NN, jnp.int32)])
    zeros = jnp.zeros((NPAD, WPAIR), jnp.float32)
    ones = jnp.ones((CHUNK, LANE), jnp.float32)

    deg_a, deg_b = _deg_kernel()(dst, zeros, ones)

    xp = jnp.pad(x, ((0, 0), (0, 1)))
    w0p = jnp.pad(W0, ((0, 1), (0, 0)))
    Ws = [w0p, W1, W2, W3]
    douts = [32, 64, 128, 256]

    g = _first_layer_call(douts[0])(xp, Ws[0], deg_a, deg_b)
    for l in range(4):
        dout = douts[l]
        tall = tuple(c.reshape(4 * NPAD, WPAIR) for c in g)
        S = _gather_scatter_kernel(dout // LANE)(src, dst, zeros, *tall)
        y, st = _combine_call(dout, len(S))(*S, *g, deg_a, deg_b)
        if l < 3:
            g = _mid_layer_call(dout, douts[l + 1])(
                y, st, Ws[l + 1], deg_a, deg_b)
    return _final_call(douts[3])(y, st)


def kernel(x, edge_index, W0, b0, W1, b1, W2, b2, W3, b3):
    return _run(x, edge_index, W0, W1, W2, W3)
